# baseline (speedup 1.0000x reference)
import functools
import jax
import jax.numpy as jnp
from jax import lax
from jax.experimental import pallas as pl
from jax.experimental.pallas import tpu as pltpu
from jax.experimental.pallas import tpu_sc as plsc

_L = 16
_BB = 4
_NB1 = 5


def _sc_lookup_body(col_hbm, row_hbm, pos_hbm, colbuf, rowbuf, chunk, sem):
    s = lax.axis_index("s")
    c = lax.axis_index("c")
    w = s * 2 + c
    pltpu.sync_copy(col_hbm.at[pl.ds(0, 32)], colbuf)
    pltpu.sync_copy(row_hbm.at[w], rowbuf)

    def body(i, _):
        for j in range(128 // _L):
            chunk[i, _L * j:_L * (j + 1)] = colbuf[i, _L * j:_L * (j + 1)]
            chunk[i, 128 + _L * j:128 + _L * (j + 1)] = rowbuf[_L * j:_L * (j + 1)]
        return 0

    lax.fori_loop(0, 32, body, 0)
    pltpu.sync_copy(chunk, pos_hbm.at[pl.ds(w * 32, 32), :])


def _tc_build_body(col_ref, row_ref, out_ref):
    BB, HW, D = out_ref.shape
    col = col_ref[...]
    row = row_ref[...]
    left = jnp.broadcast_to(col[None, :, :], (32, 32, 128)).reshape(HW, 128)
    right = jnp.broadcast_to(row[:, None, :], (32, 32, 128)).reshape(HW, 128)
    pos = jnp.concatenate([left, right], axis=-1)
    out_ref[...] = jnp.broadcast_to(pos[None], (BB, HW, D))


def _tc_fill_body(prev_ref, pos_ref, out_ref):
    BB, HW, D = out_ref.shape
    out_ref[...] = jnp.broadcast_to(pos_ref[...][None], (BB, HW, D))


def kernel(x, h, w, row_embed, col_embed):
    B, HW, D = x.shape

    mesh = plsc.VectorSubcoreMesh(core_axis_name="c", subcore_axis_name="s")
    sc_lookup = functools.partial(
        pl.kernel,
        mesh=mesh,
        out_type=jax.ShapeDtypeStruct((HW, D), jnp.float32),
        scratch_types=[
            pltpu.VMEM((32, 128), jnp.float32),
            pltpu.VMEM((128,), jnp.float32),
            pltpu.VMEM((32, 256), jnp.float32),
            pltpu.SemaphoreType.DMA,
        ],
    )(_sc_lookup_body)
    pos = sc_lookup(col_embed, row_embed)

    out1 = pl.pallas_call(
        _tc_build_body,
        grid=(_NB1,),
        in_specs=[
            pl.BlockSpec((32, 128), lambda b: (0, 0)),
            pl.BlockSpec((32, 128), lambda b: (0, 0)),
        ],
        out_specs=pl.BlockSpec((_BB, HW, D), lambda b: (b, 0, 0)),
        out_shape=jax.ShapeDtypeStruct((B, HW, D), jnp.float32),
    )(col_embed, row_embed)

    out = pl.pallas_call(
        _tc_fill_body,
        grid=(B // _BB - _NB1,),
        in_specs=[
            pl.BlockSpec((1, 8, 128), lambda b: (0, 0, 0)),
            pl.BlockSpec((HW, D), lambda b: (0, 0)),
        ],
        out_specs=pl.BlockSpec((_BB, HW, D), lambda b: (b + _NB1, 0, 0)),
        out_shape=jax.ShapeDtypeStruct((B, HW, D), jnp.float32),
        input_output_aliases={0: 0},
    )(out1, pos)
    return out
